# baseline (speedup 1.0000x reference)
import functools

import jax
import jax.numpy as jnp
import numpy as np
from jax.experimental import pallas as pl
from jax.experimental.pallas import tpu as pltpu

B, L, D = 1, 2048, 1024
NH, NKV = 16, 4
HD = D // NH
GROUPS = NH // NKV
WINDOW = 512
SINK = 4
THETA = 10000.0

BQ = 256
BK = 256
NQ = L // BQ
NKB = L // BK


def _rope_tables(n_heads):
    pos = jnp.arange(L, dtype=jnp.float32)
    inv_freq = 1.0 / (THETA ** (jnp.arange(0, HD, 2, dtype=jnp.float32) / HD))
    fr = pos[:, None] * inv_freq[None, :]
    cos = jnp.concatenate([jnp.cos(fr), jnp.cos(fr)], axis=-1)
    sin = jnp.concatenate([jnp.sin(fr), jnp.sin(fr)], axis=-1)
    return jnp.tile(cos, (1, n_heads)), jnp.tile(sin, (1, n_heads))


def _rot_weights(w, n_heads):
    w3 = w.reshape(w.shape[0], n_heads, HD)
    w1, w2 = w3[..., : HD // 2], w3[..., HD // 2 :]
    wr = jnp.concatenate([-w2, w1], axis=-1)
    return wr.reshape(w.shape[0], n_heads * HD)


def _qkv_kernel(hs_ref, wcat_ref, cq_ref, sq_ref, ck_ref, sk_ref,
                q_ref, k_ref, v_ref):
    y = jnp.dot(hs_ref[...], wcat_ref[...], preferred_element_type=jnp.float32)
    QW = NH * HD
    KW = NKV * HD
    yq = y[:, :QW]
    yqr = y[:, QW : 2 * QW]
    yk = y[:, 2 * QW : 2 * QW + KW]
    ykr = y[:, 2 * QW + KW : 2 * QW + 2 * KW]
    yv = y[:, 2 * QW + 2 * KW :]
    q_ref[...] = yq * cq_ref[...] + yqr * sq_ref[...]
    k_ref[...] = yk * ck_ref[...] + ykr * sk_ref[...]
    v_ref[...] = yv


def _qkv_call(hs, wcat, cq, sq, ck, sk):
    QW, KW = NH * HD, NKV * HD
    return pl.pallas_call(
        _qkv_kernel,
        grid=(NQ,),
        in_specs=[
            pl.BlockSpec((BQ, D), lambda i: (i, 0)),
            pl.BlockSpec((D, 2 * QW + 3 * KW), lambda i: (0, 0)),
            pl.BlockSpec((BQ, QW), lambda i: (i, 0)),
            pl.BlockSpec((BQ, QW), lambda i: (i, 0)),
            pl.BlockSpec((BQ, KW), lambda i: (i, 0)),
            pl.BlockSpec((BQ, KW), lambda i: (i, 0)),
        ],
        out_specs=[
            pl.BlockSpec((BQ, QW), lambda i: (i, 0)),
            pl.BlockSpec((BQ, KW), lambda i: (i, 0)),
            pl.BlockSpec((BQ, KW), lambda i: (i, 0)),
        ],
        out_shape=[
            jax.ShapeDtypeStruct((L, QW), jnp.float32),
            jax.ShapeDtypeStruct((L, KW), jnp.float32),
            jax.ShapeDtypeStruct((L, KW), jnp.float32),
        ],
        compiler_params=pltpu.CompilerParams(
            dimension_semantics=("arbitrary",)),
    )(hs, wcat, cq, sq, ck, sk)


def _silu(x):
    return x * jax.nn.sigmoid(x)


def _shift_pair(h):
    z = jnp.zeros((1, h.shape[1]), dtype=h.dtype)
    hp = jnp.concatenate([z, h[:-1, :]], axis=0)
    hn = jnp.concatenate([h[1:, :], z], axis=0)
    return hp, hn


def _router_kernel(xp_ref, b1w_ref, b2w_ref, b3w_ref, bpw_ref,
                   b1_ref, b2_ref, b3_ref, pb_ref, sel_ref):
    x0 = xp_ref[0:L, :]
    x1 = xp_ref[1 : L + 1, :]
    x2 = xp_ref[2 : L + 2, :]
    f32 = jnp.float32
    h = (jnp.dot(x0, b1w_ref[0], preferred_element_type=f32)
         + jnp.dot(x1, b1w_ref[1], preferred_element_type=f32)
         + jnp.dot(x2, b1w_ref[2], preferred_element_type=f32)
         + b1_ref[...])
    h = _silu(h)
    hp, hn = _shift_pair(h)
    h = (jnp.dot(hp, b2w_ref[0], preferred_element_type=f32)
         + jnp.dot(h, b2w_ref[1], preferred_element_type=f32)
         + jnp.dot(hn, b2w_ref[2], preferred_element_type=f32)
         + b2_ref[...])
    h = _silu(h)
    hp, hn = _shift_pair(h)
    h = (jnp.dot(hp, b3w_ref[0], preferred_element_type=f32)
         + jnp.dot(h, b3w_ref[1], preferred_element_type=f32)
         + jnp.dot(hn, b3w_ref[2], preferred_element_type=f32)
         + b3_ref[...])
    h = _silu(h)
    logits = jnp.dot(h, bpw_ref[...], preferred_element_type=f32) + pb_ref[...]
    sel_ref[...] = jnp.where(logits > 0.0, 1.0, 0.0)


def _router_call(xf_pad, b1w, b2w, b3w, bpw, b1, b2, b3, pb):
    return pl.pallas_call(
        _router_kernel,
        out_shape=jax.ShapeDtypeStruct((L, 128), jnp.float32),
    )(xf_pad, b1w, b2w, b3w, bpw, b1, b2, b3, pb)


def _attn_kernel(q_ref, k_ref, v_ref, sel_ref, wo_ref, out_ref):
    qi = pl.program_id(0)
    h = pl.program_id(1)
    g = h // GROUPS
    scale = 1.0 / np.sqrt(HD)
    q = q_ref[0] * scale

    def body(kj, carry):
        m, l, acc = carry
        kb = k_ref[g, pl.ds(kj * BK, BK), :]
        vb = v_ref[g, pl.ds(kj * BK, BK), :]
        s = jax.lax.dot_general(q, kb, (((1,), (1,)), ((), ())),
                                preferred_element_type=jnp.float32)
        ii = qi * BQ + jax.lax.broadcasted_iota(jnp.int32, (BQ, BK), 0)
        jj = kj * BK + jax.lax.broadcasted_iota(jnp.int32, (BQ, BK), 1)
        selb = sel_ref[g, :, pl.ds(kj * BK, BK)] > 0.0
        mask = (jj <= ii) & (((ii - jj) < WINDOW) | (jj < SINK) | selb)
        s = jnp.where(mask, s, -1e30)
        m_new = jnp.maximum(m, jnp.max(s, axis=1, keepdims=True))
        alpha = jnp.exp(m - m_new)
        p = jnp.exp(s - m_new)
        l_new = l * alpha + jnp.sum(p, axis=1, keepdims=True)
        acc_new = acc * alpha + jax.lax.dot_general(
            p, vb, (((1,), (0,)), ((), ())), preferred_element_type=jnp.float32)
        return m_new, l_new, acc_new

    m0 = jnp.full((BQ, 1), -1e30, dtype=jnp.float32)
    l0 = jnp.zeros((BQ, 1), dtype=jnp.float32)
    a0 = jnp.zeros((BQ, HD), dtype=jnp.float32)
    m, l, acc = jax.lax.fori_loop(0, qi + 1, body, (m0, l0, a0))
    o = acc / l

    @pl.when(h == 0)
    def _():
        out_ref[...] = jnp.zeros_like(out_ref)

    wo_h = wo_ref[pl.ds(h * HD, HD), :]
    out_ref[...] += jnp.dot(o, wo_h, preferred_element_type=jnp.float32)


def _attn_call(qh, kh, vh, selr, wo):
    return pl.pallas_call(
        _attn_kernel,
        grid=(NQ, NH),
        in_specs=[
            pl.BlockSpec((1, BQ, HD), lambda qi, h: (h, qi, 0)),
            pl.BlockSpec((NKV, L, HD), lambda qi, h: (0, 0, 0)),
            pl.BlockSpec((NKV, L, HD), lambda qi, h: (0, 0, 0)),
            pl.BlockSpec((NKV, 1, L), lambda qi, h: (0, 0, 0)),
            pl.BlockSpec((D, D), lambda qi, h: (0, 0)),
        ],
        out_specs=pl.BlockSpec((BQ, D), lambda qi, h: (qi, 0)),
        out_shape=jax.ShapeDtypeStruct((L, D), jnp.float32),
        compiler_params=pltpu.CompilerParams(
            dimension_semantics=("parallel", "arbitrary")),
    )(qh, kh, vh, selr, wo)


@jax.jit
def _run(hidden_states, Wq, Wk, Wv, Wo, conv1_w, conv1_b, conv2_w, conv2_b,
         conv3_w, conv3_b, proj_w, proj_b):
    hs = hidden_states[0]
    QW, KW = NH * HD, NKV * HD

    wqr = _rot_weights(Wq, NH)
    wkr = _rot_weights(Wk, NKV)
    wcat = jnp.concatenate([Wq, wqr, Wk, wkr, Wv], axis=1)
    cq, sq = _rope_tables(NH)
    ck, sk = _rope_tables(NKV)

    q, k, v = _qkv_call(hs, wcat, cq, sq, ck, sk)

    xf = jnp.concatenate(
        [k.reshape(L, NKV, HD), v.reshape(L, NKV, HD)], axis=-1
    ).reshape(L, 2 * KW)
    xf_pad = jnp.zeros((L + 8, 2 * KW), jnp.float32).at[1 : L + 1].set(xf)

    cin1, cout1 = 2 * HD, HD
    b1w = jnp.zeros((3, NKV * cin1, NKV * cout1), jnp.float32)
    b2w = jnp.zeros((3, NKV * cout1, NKV * cout1 // 2), jnp.float32)
    b3w = jnp.zeros((3, NKV * cout1 // 2, NKV * cout1 // 4), jnp.float32)
    bpw = jnp.zeros((NKV * cout1 // 4, 128), jnp.float32)
    for g in range(NKV):
        b1w = b1w.at[:, g * cin1 : (g + 1) * cin1,
                     g * cout1 : (g + 1) * cout1].set(
            jnp.transpose(conv1_w[g * cout1 : (g + 1) * cout1, :, 0, :],
                          (2, 1, 0)))
        b2w = b2w.at[:, g * 64 : (g + 1) * 64, g * 32 : (g + 1) * 32].set(
            jnp.transpose(conv2_w[g * 32 : (g + 1) * 32, :, 0, :], (2, 1, 0)))
        b3w = b3w.at[:, g * 32 : (g + 1) * 32, g * 16 : (g + 1) * 16].set(
            jnp.transpose(conv3_w[g * 16 : (g + 1) * 16, :, 0, :], (2, 1, 0)))
        bpw = bpw.at[g * 16 : (g + 1) * 16, g].set(proj_w[g, :, 0, 0])
    b1 = conv1_b[None, :]
    b2 = conv2_b[None, :]
    b3 = conv3_b[None, :]
    pb = jnp.zeros((1, 128), jnp.float32).at[0, :NKV].set(proj_b)

    sel = _router_call(xf_pad, b1w, b2w, b3w, bpw, b1, b2, b3, pb)
    selr = sel[:, :NKV].T.reshape(NKV, 1, L)

    qh = q.reshape(L, NH, HD).transpose(1, 0, 2)
    kh = k.reshape(L, NKV, HD).transpose(1, 0, 2)
    vh = v.reshape(L, NKV, HD).transpose(1, 0, 2)

    out = _attn_call(qh, kh, vh, selr, Wo)
    return out[None]


def kernel(hidden_states, Wq, Wk, Wv, Wo, conv1_w, conv1_b, conv2_w, conv2_b,
           conv3_w, conv3_b, proj_w, proj_b):
    return _run(hidden_states, Wq, Wk, Wv, Wo, conv1_w, conv1_b, conv2_w,
                conv2_b, conv3_w, conv3_b, proj_w, proj_b)
